# baseline (speedup 1.0000x reference)
import functools

import jax
import jax.numpy as jnp
from jax import lax
from jax.experimental import pallas as pl
from jax.experimental.pallas import tpu as pltpu
from jax.experimental.pallas import tpu_sc as plsc

N_EDGES = 160000
D_FEAT = 256
N_NODES = 10000

NC = 2
NS = 16
CHUNK = 128
NCHUNKS = N_EDGES // CHUNK
CPT = 80
HALF = D_FEAT // NC
N_PAD = 10112
RPT = N_PAD // NS
LAST_RPT = N_NODES - (NS - 1) * RPT
NBUF = 2
K = 16
NV = HALF // 16


def _sc_segment_sum(i, idx2, zrows):
    mesh = plsc.VectorSubcoreMesh(core_axis_name="c", subcore_axis_name="s")

    @functools.partial(
        pl.kernel,
        out_type=jax.ShapeDtypeStruct((N_NODES, D_FEAT), jnp.float32),
        mesh=mesh,
        scratch_types=[
            pltpu.VMEM((CPT, CHUNK), jnp.int32),
            [pltpu.VMEM((CHUNK, HALF), jnp.float32) for _ in range(NBUF)],
            pltpu.VMEM((K, HALF), jnp.float32),
            pltpu.VMEM((K,), jnp.int32),
            pltpu.VMEM_SHARED((N_PAD, HALF), jnp.float32),
            [pltpu.SemaphoreType.DMA for _ in range(NBUF)],
            pltpu.SemaphoreType.DMA,
            pltpu.SemaphoreType.DMA,
            pltpu.SemaphoreType.DMA,
        ],
    )
    def k(i_hbm, idx_hbm, z_hbm, out_hbm, idx_v, bufs, cbuf, cidx, accum,
          gsems, rsem, csem, zsem):
        cc = lax.axis_index("c")
        s = lax.axis_index("s")
        base = s * CPT
        n = jnp.minimum(CPT, NCHUNKS - base)
        trash = N_NODES + s * 7 + lax.iota(jnp.int32, 16) % 7

        def gslice(c):
            return i_hbm.at[pl.ds(c * CHUNK, CHUNK), pl.ds(cc * HALF, HALF)]

        idx_cp = pltpu.async_copy(idx_hbm.at[pl.ds(base, CPT)], idx_v, zsem)
        for b in range(NBUF):
            pltpu.async_copy(gslice(base + b), bufs[b], gsems[b])
        for t in range(4):
            pltpu.sync_copy(z_hbm,
                            accum.at[pl.ds(s * RPT + t * CHUNK, CHUNK)])
        pltpu.sync_copy(z_hbm.at[pl.ds(0, RPT - 4 * CHUNK)],
                        accum.at[pl.ds(s * RPT + 4 * CHUNK, RPT - 4 * CHUNK)])
        idx_cp.wait()
        plsc.subcore_barrier()

        def body(t, carry):
            j0 = t * 2
            j1 = j0 + 1

            @pl.when(j0 < n)
            def _():
                pltpu.make_async_copy(gslice(base + j0), bufs[0],
                                      gsems[0]).wait()
                pltpu.async_copy(bufs[0], accum.at[idx_v.at[j0]], rsem,
                                 add=True)

                pltpu.make_async_copy(gslice(base + j1), bufs[1],
                                      gsems[1]).wait()

                @pl.when(t >= 1)
                def _():
                    pltpu.make_async_copy(z_hbm.at[pl.ds(0, K)], cbuf,
                                          csem).wait()

                lane = lax.iota(jnp.int32, 16)

                def step(val, r, st):
                    slot, cur, cvec, acc = st
                    same = val == cur
                    mi = same.astype(jnp.int32)
                    mf = mi.astype(jnp.float32)
                    sl = jnp.minimum(slot, K - 1)

                    @pl.when(jnp.logical_not(same))
                    def _():
                        for v in range(NV):
                            cbuf[sl, pl.ds(v * 16, 16)] = acc[v]

                    ncvec = jnp.where(lane == sl * (1 - mi) + mi * -1,
                                      cur, cvec)
                    nacc = tuple(
                        bufs[1][r, pl.ds(v * 16, 16)] + acc[v] * mf
                        for v in range(NV))
                    return (slot + 1 - mi, val, ncvec, nacc)

                ig0 = idx_v[j1, pl.ds(0, 16)]
                st = (jnp.int32(0), ig0[0], trash,
                      tuple(bufs[1][0, pl.ds(v * 16, 16)]
                            for v in range(NV)))
                for l in range(1, 16):
                    st = step(ig0[l], l, st)

                def group(g, st):
                    ig = idx_v[j1, pl.ds(g * 16, 16)]
                    for l in range(16):
                        st = step(ig[l], g * 16 + l, st)
                    return st

                slot, cur, cvec, acc = lax.fori_loop(1, CHUNK // 16,
                                                     group, st)
                sl = jnp.minimum(slot, K - 1)
                for v in range(NV):
                    cbuf[sl, pl.ds(v * 16, 16)] = acc[v]
                cvec = jnp.where(lane == sl, cur, cvec)
                ovf = slot > K - 1
                oi = ovf.astype(jnp.int32)
                cidx[pl.ds(0, 16)] = trash * oi + cvec * (1 - oi)

                @pl.when(ovf)
                def _():
                    pltpu.sync_copy(bufs[1], accum.at[idx_v.at[j1]],
                                    add=True)

                pltpu.async_copy(cbuf, accum.at[cidx], csem, add=True)

                pltpu.make_async_copy(z_hbm, bufs[0], rsem).wait()

                @pl.when(j0 + 2 < n)
                def _():
                    pltpu.async_copy(gslice(base + j0 + 2), bufs[0],
                                     gsems[0])
                    pltpu.async_copy(gslice(base + j1 + 2), bufs[1],
                                     gsems[1])

            return carry

        lax.fori_loop(0, CPT // 2, body, 0)
        pltpu.make_async_copy(z_hbm.at[pl.ds(0, K)], cbuf, csem).wait()
        plsc.subcore_barrier()

        @pl.when(s < NS - 1)
        def _full_copy():
            pltpu.sync_copy(
                accum.at[pl.ds(s * RPT, RPT)],
                out_hbm.at[pl.ds(s * RPT, RPT), pl.ds(cc * HALF, HALF)],
            )

        @pl.when(s == NS - 1)
        def _last_copy():
            pltpu.sync_copy(
                accum.at[pl.ds((NS - 1) * RPT, LAST_RPT)],
                out_hbm.at[pl.ds((NS - 1) * RPT, LAST_RPT),
                           pl.ds(cc * HALF, HALF)],
            )

    return k(i, idx2, zrows)


@jax.jit
def kernel(i, idx_i):
    pad = NS * CPT * CHUNK - N_EDGES
    idx2 = jnp.pad(idx_i, (0, pad)).reshape(NS * CPT, CHUNK)
    zrows = jnp.zeros((CHUNK, HALF), jnp.float32)
    return _sc_segment_sum(i, idx2, zrows)


# speedup vs baseline: 1.2930x; 1.2930x over previous
import functools

import jax
import jax.numpy as jnp
from jax import lax
from jax.experimental import pallas as pl
from jax.experimental.pallas import tpu as pltpu
from jax.experimental.pallas import tpu_sc as plsc

N_EDGES = 160000
D_FEAT = 256
N_NODES = 10000

NC = 2
NS = 16
CHUNK = 128
NCHUNKS = N_EDGES // CHUNK
CPT = 80
HALF = D_FEAT // NC
N_PAD = 10240
RPT = N_PAD // NS
LAST_RPT = N_NODES - (NS - 1) * RPT
NBUF = 2


def _sc_segment_sum(i, idx2, zrows):
    mesh = plsc.VectorSubcoreMesh(core_axis_name="c", subcore_axis_name="s")

    @functools.partial(
        pl.kernel,
        out_type=jax.ShapeDtypeStruct((N_NODES, D_FEAT), jnp.float32),
        mesh=mesh,
        scratch_types=[
            pltpu.VMEM((CPT, CHUNK), jnp.int32),
            [pltpu.VMEM((CHUNK, HALF), jnp.float32) for _ in range(NBUF)],
            pltpu.VMEM_SHARED((N_PAD, HALF), jnp.float32),
            [pltpu.SemaphoreType.DMA for _ in range(NBUF)],
            [pltpu.SemaphoreType.DMA for _ in range(NBUF)],
            pltpu.SemaphoreType.DMA,
        ],
    )
    def k(i_hbm, idx_hbm, z_hbm, out_hbm, idx_v, bufs, accum, gsems, ssems,
          zsem):
        cc = lax.axis_index("c")
        s = lax.axis_index("s")
        base = s * CPT
        n = jnp.minimum(CPT, NCHUNKS - base)

        def gslice(c):
            return i_hbm.at[pl.ds(c * CHUNK, CHUNK), pl.ds(cc * HALF, HALF)]

        idx_cp = pltpu.async_copy(idx_hbm.at[pl.ds(base, CPT)], idx_v, zsem)
        for b in range(NBUF):
            pltpu.async_copy(gslice(base + b), bufs[b], gsems[b])
        zcps = [
            pltpu.async_copy(z_hbm, accum.at[pl.ds(s * RPT + t * CHUNK, CHUNK)],
                             ssems[0])
            for t in range(RPT // CHUNK)
        ]
        for z in zcps:
            z.wait()
        idx_cp.wait()
        plsc.subcore_barrier()

        def body(j2, carry):
            for b in range(NBUF):
                j = j2 * NBUF + b
                c = base + j

                @pl.when(j < n)
                def _():
                    pltpu.make_async_copy(gslice(c), bufs[b], gsems[b]).wait()
                    pltpu.sync_copy(bufs[b], accum.at[idx_v.at[j]], add=True)

                    @pl.when(j + NBUF < n)
                    def _():
                        pltpu.async_copy(gslice(c + NBUF), bufs[b], gsems[b])

            return carry

        lax.fori_loop(0, CPT // NBUF, body, 0)
        plsc.subcore_barrier()

        @pl.when(s < NS - 1)
        def _full_copy():
            pltpu.sync_copy(
                accum.at[pl.ds(s * RPT, RPT)],
                out_hbm.at[pl.ds(s * RPT, RPT), pl.ds(cc * HALF, HALF)],
            )

        @pl.when(s == NS - 1)
        def _last_copy():
            pltpu.sync_copy(
                accum.at[pl.ds((NS - 1) * RPT, LAST_RPT)],
                out_hbm.at[pl.ds((NS - 1) * RPT, LAST_RPT),
                           pl.ds(cc * HALF, HALF)],
            )

    return k(i, idx2, zrows)


@jax.jit
def kernel(i, idx_i):
    pad = NS * CPT * CHUNK - N_EDGES
    idx2 = jnp.pad(idx_i, (0, pad)).reshape(NS * CPT, CHUNK)
    zrows = jnp.zeros((CHUNK, HALF), jnp.float32)
    return _sc_segment_sum(i, idx2, zrows)
